# baseline (speedup 1.0000x reference)
import functools

import jax
import jax.numpy as jnp
from jax import lax
from jax.experimental import pallas as pl
from jax.experimental.pallas import tpu as pltpu
from jax.experimental.pallas import tpu_sc as plsc

N = 10000
E = 320000
C = 128
OUTC = 64

NC = 2
NS = 16
NW = NC * NS
K = 128
NCHUNK = 80
EPW = NCHUNK * K
E_PAD = NW * EPW
TOT_CH = E_PAD // K
CH_A = 56
CH_B = 104
SG = 8
N_PAD = 10112
RPT = N_PAD // NS

_mesh = plsc.VectorSubcoreMesh(
    core_axis_name="c", subcore_axis_name="s", num_cores=NC, num_subcores=NS
)


@functools.partial(
    pl.kernel,
    out_type=jax.ShapeDtypeStruct((NC, N_PAD, C), jnp.float32),
    mesh=_mesh,
    scratch_types=[
        pltpu.VMEM((NCHUNK, K), jnp.int32),
        pltpu.VMEM((K, C), jnp.float32),
        pltpu.VMEM_SHARED((N_PAD, C), jnp.float32),
        pltpu.SemaphoreType.DMA,
    ],
)
def _sc_degree(col_hbm, ones_hbm, zeros_hbm, out_hbm, colv, onesv, acc, dsem):
    c = lax.axis_index("c")
    s = lax.axis_index("s")
    w = c * NS + s
    pltpu.sync_copy(zeros_hbm, acc.at[pl.ds(s * RPT, RPT)])
    pltpu.sync_copy(col_hbm.at[pl.ds(w * NCHUNK, NCHUNK)], colv)
    pltpu.sync_copy(ones_hbm, onesv)
    plsc.subcore_barrier()

    def body(j, carry):
        pltpu.async_copy(onesv, acc.at[colv.at[j]], dsem, add=True)
        return carry

    lax.fori_loop(0, NCHUNK, body, 0)

    def drain(j, carry):
        pltpu.make_async_copy(onesv, acc.at[colv.at[j]], dsem).wait()
        return carry

    lax.fori_loop(0, NCHUNK, drain, 0)
    plsc.subcore_barrier()
    pltpu.sync_copy(acc.at[pl.ds(s * RPT, RPT)], out_hbm.at[c, pl.ds(s * RPT, RPT)])


@functools.partial(
    pl.kernel,
    out_type=jax.ShapeDtypeStruct((NC, N_PAD, C), jnp.float32),
    mesh=_mesh,
    scratch_types=[
        pltpu.VMEM((SG, K), jnp.int32),
        pltpu.VMEM((SG, K), jnp.int32),
        pltpu.VMEM_SHARED((N_PAD, C), jnp.float32),
        pltpu.VMEM((K, C), jnp.float32),
        pltpu.SemaphoreType.DMA,
    ],
)
def _sc_scatter(g_hbm, row_hbm, col_hbm, zeros_hbm, out_hbm,
                rowv, colv, acc, buf, sem):
    c = lax.axis_index("c")
    s = lax.axis_index("s")
    pltpu.sync_copy(zeros_hbm, acc.at[pl.ds(s * RPT, RPT)])
    plsc.subcore_barrier()

    start = jnp.where(c == 0, s * CH_A, NS * CH_A + s * CH_B)
    nst = jnp.where(c == 0, CH_A // SG, CH_B // SG)

    def stage(h, carry):
        base = start + h * SG
        pltpu.sync_copy(row_hbm.at[pl.ds(base, SG)], rowv)
        pltpu.sync_copy(col_hbm.at[pl.ds(base, SG)], colv)
        for b in range(SG):
            pltpu.async_copy(g_hbm.at[rowv.at[b]], buf, sem).wait()
            pltpu.sync_copy(buf, acc.at[colv.at[b]], add=True)
        return carry

    lax.fori_loop(0, nst, stage, 0)
    plsc.subcore_barrier()
    pltpu.sync_copy(acc.at[pl.ds(s * RPT, RPT)], out_hbm.at[c, pl.ds(s * RPT, RPT)])


BLK = 1000
NBLK = N // BLK


def _dinv(deg_ref):
    deg = deg_ref[0, :, :1] + deg_ref[1, :, :1] + 1.0
    return lax.rsqrt(deg)


def _tc_first(x_ref, wp_ref, bp_ref, w0_ref, deg_ref, g_ref):
    dinv = _dinv(deg_ref)
    h = jnp.dot(x_ref[...], wp_ref[...].T, preferred_element_type=jnp.float32)
    h = h + bp_ref[...]
    g_ref[...] = dinv * jnp.dot(h, w0_ref[...].T, preferred_element_type=jnp.float32)


def _tc_mid(acc_ref, g_ref, deg_ref, b_ref, w_ref, o_ref):
    dinv = _dinv(deg_ref)
    tot = acc_ref[0] + acc_ref[1] + g_ref[...]
    h = jnp.maximum(dinv * tot + b_ref[...], 0.0)
    o_ref[...] = dinv * jnp.dot(h, w_ref[...].T, preferred_element_type=jnp.float32)


def _tc_final(acc_ref, g_ref, deg_ref, b_ref, wo_ref, bo_ref, o_ref):
    dinv = _dinv(deg_ref)
    tot = acc_ref[0] + acc_ref[1] + g_ref[...]
    h = jnp.maximum(dinv * tot + b_ref[...], 0.0)
    o_ref[...] = jnp.dot(h, wo_ref[...].T, preferred_element_type=jnp.float32)
    o_ref[...] += bo_ref[...]


def _row_spec(width):
    return pl.BlockSpec((BLK, width), lambda i: (i, 0))


def _bcast_spec(shape):
    nd = len(shape)
    return pl.BlockSpec(shape, lambda i, _n=nd: (0,) * _n)


_DEG_SPEC = pl.BlockSpec((NC, BLK, C), lambda i: (0, i, 0))
_ACC_SPEC = pl.BlockSpec((NC, BLK, C), lambda i: (0, i, 0))


def kernel(x, edge_index, Wp, bp, W0, b0, W1, b1, W2, b2, Wo, bo):
    row = edge_index[0].astype(jnp.int32)
    col = edge_index[1].astype(jnp.int32)
    npad = E_PAD - E
    rowp = jnp.concatenate([row, jnp.zeros((npad,), jnp.int32)]).reshape(TOT_CH, K)
    colp = jnp.concatenate([col, jnp.full((npad,), N, jnp.int32)]).reshape(TOT_CH, K)
    onesC = jnp.ones((K, C), jnp.float32)
    zerosC = jnp.zeros((RPT, C), jnp.float32)
    bp2 = bp.reshape(1, C)
    b0_2 = b0.reshape(1, C)
    b1_2 = b1.reshape(1, C)
    b2_2 = b2.reshape(1, C)
    bo2 = bo.reshape(1, OUTC)

    degp = _sc_degree(colp, onesC, zerosC)

    g0 = pl.pallas_call(
        _tc_first,
        grid=(NBLK,),
        in_specs=[
            _row_spec(C),
            _bcast_spec((C, C)),
            _bcast_spec((1, C)),
            _bcast_spec((C, C)),
            _DEG_SPEC,
        ],
        out_specs=_row_spec(C),
        out_shape=jax.ShapeDtypeStruct((N, C), jnp.float32),
    )(x, Wp, bp2, W0, degp)

    g = g0
    for Wn, bn in ((W1, b0_2), (W2, b1_2)):
        accp = _sc_scatter(g, rowp, colp, zerosC)
        g = pl.pallas_call(
            _tc_mid,
            grid=(NBLK,),
            in_specs=[
                _ACC_SPEC,
                _row_spec(C),
                _DEG_SPEC,
                _bcast_spec((1, C)),
                _bcast_spec((C, C)),
            ],
            out_specs=_row_spec(C),
            out_shape=jax.ShapeDtypeStruct((N, C), jnp.float32),
        )(accp, g, degp, bn, Wn)

    accp = _sc_scatter(g, rowp, colp, zerosC)
    out = pl.pallas_call(
        _tc_final,
        grid=(NBLK,),
        in_specs=[
            _ACC_SPEC,
            _row_spec(C),
            _DEG_SPEC,
            _bcast_spec((1, C)),
            _bcast_spec((OUTC, C)),
            _bcast_spec((1, OUTC)),
        ],
        out_specs=_row_spec(OUTC),
        out_shape=jax.ShapeDtypeStruct((N, OUTC), jnp.float32),
    )(accp, g, degp, b2_2, Wo, bo2)
    return out


# speedup vs baseline: 1.1683x; 1.1683x over previous
import functools

import jax
import jax.numpy as jnp
from jax import lax
from jax.experimental import pallas as pl
from jax.experimental.pallas import tpu as pltpu
from jax.experimental.pallas import tpu_sc as plsc

N = 10000
E = 320000
C = 128
OUTC = 64

NC = 2
NS = 16
NW = NC * NS
K = 128
NCHUNK = 80
EPW = NCHUNK * K
E_PAD = NW * EPW
TOT_CH = E_PAD // K
CH_A = 104
CH_B = 56
SG = 8
N_PAD = 10112
RPT = N_PAD // NS

_mesh = plsc.VectorSubcoreMesh(
    core_axis_name="c", subcore_axis_name="s", num_cores=NC, num_subcores=NS
)


@functools.partial(
    pl.kernel,
    out_type=jax.ShapeDtypeStruct((NC, N_PAD, C), jnp.float32),
    mesh=_mesh,
    scratch_types=[
        pltpu.VMEM((NCHUNK, K), jnp.int32),
        pltpu.VMEM((K, C), jnp.float32),
        pltpu.VMEM_SHARED((N_PAD, C), jnp.float32),
        pltpu.SemaphoreType.DMA,
    ],
)
def _sc_degree(col_hbm, ones_hbm, zeros_hbm, out_hbm, colv, onesv, acc, dsem):
    c = lax.axis_index("c")
    s = lax.axis_index("s")
    w = c * NS + s
    pltpu.sync_copy(zeros_hbm, acc.at[pl.ds(s * RPT, RPT)])
    pltpu.sync_copy(col_hbm.at[pl.ds(w * NCHUNK, NCHUNK)], colv)
    pltpu.sync_copy(ones_hbm, onesv)
    plsc.subcore_barrier()

    def body(j, carry):
        pltpu.async_copy(onesv, acc.at[colv.at[j]], dsem, add=True)
        return carry

    lax.fori_loop(0, NCHUNK, body, 0)

    def drain(j, carry):
        pltpu.make_async_copy(onesv, acc.at[colv.at[j]], dsem).wait()
        return carry

    lax.fori_loop(0, NCHUNK, drain, 0)
    plsc.subcore_barrier()
    pltpu.sync_copy(acc.at[pl.ds(s * RPT, RPT)], out_hbm.at[c, pl.ds(s * RPT, RPT)])


@functools.partial(
    pl.kernel,
    out_type=jax.ShapeDtypeStruct((NC, N_PAD, C), jnp.float32),
    mesh=_mesh,
    scratch_types=[
        pltpu.VMEM((SG, K), jnp.int32),
        pltpu.VMEM((SG, K), jnp.int32),
        pltpu.VMEM_SHARED((N_PAD, C), jnp.float32),
        pltpu.VMEM((K, C), jnp.float32),
        pltpu.SemaphoreType.DMA,
    ],
)
def _sc_scatter(g_hbm, row_hbm, col_hbm, zeros_hbm, out_hbm,
                rowv, colv, acc, buf, sem):
    c = lax.axis_index("c")
    s = lax.axis_index("s")
    pltpu.sync_copy(zeros_hbm, acc.at[pl.ds(s * RPT, RPT)])
    plsc.subcore_barrier()

    start = jnp.where(c == 0, s * CH_A, NS * CH_A + s * CH_B)
    nst = jnp.where(c == 0, CH_A // SG, CH_B // SG)

    def stage(h, carry):
        base = start + h * SG
        pltpu.sync_copy(row_hbm.at[pl.ds(base, SG)], rowv)
        pltpu.sync_copy(col_hbm.at[pl.ds(base, SG)], colv)
        for b in range(SG):
            pltpu.async_copy(g_hbm.at[rowv.at[b]], buf, sem).wait()
            pltpu.sync_copy(buf, acc.at[colv.at[b]], add=True)
        return carry

    lax.fori_loop(0, nst, stage, 0)
    plsc.subcore_barrier()
    pltpu.sync_copy(acc.at[pl.ds(s * RPT, RPT)], out_hbm.at[c, pl.ds(s * RPT, RPT)])


BLK = 1000
NBLK = N // BLK


def _dinv(deg_ref):
    deg = deg_ref[0, :, :1] + deg_ref[1, :, :1] + 1.0
    return lax.rsqrt(deg)


def _tc_first(x_ref, wp_ref, bp_ref, w0_ref, deg_ref, g_ref):
    dinv = _dinv(deg_ref)
    h = jnp.dot(x_ref[...], wp_ref[...].T, preferred_element_type=jnp.float32)
    h = h + bp_ref[...]
    g_ref[...] = dinv * jnp.dot(h, w0_ref[...].T, preferred_element_type=jnp.float32)


def _tc_mid(acc_ref, g_ref, deg_ref, b_ref, w_ref, o_ref):
    dinv = _dinv(deg_ref)
    tot = acc_ref[0] + acc_ref[1] + g_ref[...]
    h = jnp.maximum(dinv * tot + b_ref[...], 0.0)
    o_ref[...] = dinv * jnp.dot(h, w_ref[...].T, preferred_element_type=jnp.float32)


def _tc_final(acc_ref, g_ref, deg_ref, b_ref, wo_ref, bo_ref, o_ref):
    dinv = _dinv(deg_ref)
    tot = acc_ref[0] + acc_ref[1] + g_ref[...]
    h = jnp.maximum(dinv * tot + b_ref[...], 0.0)
    o_ref[...] = jnp.dot(h, wo_ref[...].T, preferred_element_type=jnp.float32)
    o_ref[...] += bo_ref[...]


def _row_spec(width):
    return pl.BlockSpec((BLK, width), lambda i: (i, 0))


def _bcast_spec(shape):
    nd = len(shape)
    return pl.BlockSpec(shape, lambda i, _n=nd: (0,) * _n)


_DEG_SPEC = pl.BlockSpec((NC, BLK, C), lambda i: (0, i, 0))
_ACC_SPEC = pl.BlockSpec((NC, BLK, C), lambda i: (0, i, 0))


def kernel(x, edge_index, Wp, bp, W0, b0, W1, b1, W2, b2, Wo, bo):
    row = edge_index[0].astype(jnp.int32)
    col = edge_index[1].astype(jnp.int32)
    npad = E_PAD - E
    rowp = jnp.concatenate([row, jnp.zeros((npad,), jnp.int32)]).reshape(TOT_CH, K)
    colp = jnp.concatenate([col, jnp.full((npad,), N, jnp.int32)]).reshape(TOT_CH, K)
    onesC = jnp.ones((K, C), jnp.float32)
    zerosC = jnp.zeros((RPT, C), jnp.float32)
    bp2 = bp.reshape(1, C)
    b0_2 = b0.reshape(1, C)
    b1_2 = b1.reshape(1, C)
    b2_2 = b2.reshape(1, C)
    bo2 = bo.reshape(1, OUTC)

    degp = _sc_degree(colp, onesC, zerosC)

    g0 = pl.pallas_call(
        _tc_first,
        grid=(NBLK,),
        in_specs=[
            _row_spec(C),
            _bcast_spec((C, C)),
            _bcast_spec((1, C)),
            _bcast_spec((C, C)),
            _DEG_SPEC,
        ],
        out_specs=_row_spec(C),
        out_shape=jax.ShapeDtypeStruct((N, C), jnp.float32),
    )(x, Wp, bp2, W0, degp)

    g = g0
    for Wn, bn in ((W1, b0_2), (W2, b1_2)):
        accp = _sc_scatter(g, rowp, colp, zerosC)
        g = pl.pallas_call(
            _tc_mid,
            grid=(NBLK,),
            in_specs=[
                _ACC_SPEC,
                _row_spec(C),
                _DEG_SPEC,
                _bcast_spec((1, C)),
                _bcast_spec((C, C)),
            ],
            out_specs=_row_spec(C),
            out_shape=jax.ShapeDtypeStruct((N, C), jnp.float32),
        )(accp, g, degp, bn, Wn)

    accp = _sc_scatter(g, rowp, colp, zerosC)
    out = pl.pallas_call(
        _tc_final,
        grid=(NBLK,),
        in_specs=[
            _ACC_SPEC,
            _row_spec(C),
            _DEG_SPEC,
            _bcast_spec((1, C)),
            _bcast_spec((OUTC, C)),
            _bcast_spec((1, OUTC)),
        ],
        out_specs=_row_spec(OUTC),
        out_shape=jax.ShapeDtypeStruct((N, OUTC), jnp.float32),
    )(accp, g, degp, b2_2, Wo, bo2)
    return out


# speedup vs baseline: 1.1773x; 1.0077x over previous
import functools

import jax
import jax.numpy as jnp
from jax import lax
from jax.experimental import pallas as pl
from jax.experimental.pallas import tpu as pltpu
from jax.experimental.pallas import tpu_sc as plsc

N = 10000
E = 320000
C = 128
OUTC = 64

NC = 2
NS = 16
NW = NC * NS
K = 128
NCHUNK = 80
EPW = NCHUNK * K
E_PAD = NW * EPW
TOT_CH = E_PAD // K
CH_A = 104
CH_B = 56
SG = 8
N_PAD = 10112
RPT = N_PAD // NS

_mesh = plsc.VectorSubcoreMesh(
    core_axis_name="c", subcore_axis_name="s", num_cores=NC, num_subcores=NS
)


@functools.partial(
    pl.kernel,
    out_type=jax.ShapeDtypeStruct((NC, N_PAD, C), jnp.float32),
    mesh=_mesh,
    scratch_types=[
        pltpu.VMEM((NCHUNK, K), jnp.int32),
        pltpu.VMEM((K, C), jnp.float32),
        pltpu.VMEM_SHARED((N_PAD, C), jnp.float32),
        pltpu.SemaphoreType.DMA,
    ],
)
def _sc_degree(col_hbm, ones_hbm, zeros_hbm, out_hbm, colv, onesv, acc, dsem):
    c = lax.axis_index("c")
    s = lax.axis_index("s")
    w = c * NS + s
    pltpu.sync_copy(zeros_hbm, acc.at[pl.ds(s * RPT, RPT)])
    pltpu.sync_copy(col_hbm.at[pl.ds(w * NCHUNK, NCHUNK)], colv)
    pltpu.sync_copy(ones_hbm, onesv)
    plsc.subcore_barrier()

    def body(j, carry):
        pltpu.async_copy(onesv, acc.at[colv.at[j]], dsem, add=True)
        return carry

    lax.fori_loop(0, NCHUNK, body, 0)

    def drain(j, carry):
        pltpu.make_async_copy(onesv, acc.at[colv.at[j]], dsem).wait()
        return carry

    lax.fori_loop(0, NCHUNK, drain, 0)
    plsc.subcore_barrier()
    pltpu.sync_copy(acc.at[pl.ds(s * RPT, RPT)], out_hbm.at[c, pl.ds(s * RPT, RPT)])


@functools.partial(
    pl.kernel,
    out_type=jax.ShapeDtypeStruct((NC, N_PAD, C), jnp.float32),
    mesh=_mesh,
    scratch_types=[
        pltpu.VMEM((max(CH_A, CH_B), K), jnp.int32),
        pltpu.VMEM((max(CH_A, CH_B), K), jnp.int32),
        pltpu.VMEM_SHARED((N_PAD, C), jnp.float32),
        pltpu.VMEM((K, C), jnp.float32),
        pltpu.SemaphoreType.DMA,
    ],
)
def _sc_scatter(g_hbm, row_hbm, col_hbm, zeros_hbm, out_hbm,
                rowv, colv, acc, buf, sem):
    c = lax.axis_index("c")
    s = lax.axis_index("s")
    pltpu.sync_copy(zeros_hbm, acc.at[pl.ds(s * RPT, RPT)])
    plsc.subcore_barrier()

    def run(start, ch):
        pltpu.sync_copy(row_hbm.at[pl.ds(start, ch)], rowv.at[pl.ds(0, ch)])
        pltpu.sync_copy(col_hbm.at[pl.ds(start, ch)], colv.at[pl.ds(0, ch)])

        def body(j, carry):
            pltpu.async_copy(g_hbm.at[rowv.at[j]], buf, sem).wait()
            pltpu.sync_copy(buf, acc.at[colv.at[j]], add=True)
            return carry

        lax.fori_loop(0, ch, body, 0)

    @pl.when(c == 0)
    def _():
        run(s * CH_A, CH_A)

    @pl.when(c == 1)
    def _():
        run(NS * CH_A + s * CH_B, CH_B)

    plsc.subcore_barrier()
    pltpu.sync_copy(acc.at[pl.ds(s * RPT, RPT)], out_hbm.at[c, pl.ds(s * RPT, RPT)])


BLK = 1000
NBLK = N // BLK


def _dinv(deg_ref):
    deg = deg_ref[0, :, :1] + deg_ref[1, :, :1] + 1.0
    return lax.rsqrt(deg)


def _tc_first(x_ref, wp_ref, bp_ref, w0_ref, deg_ref, g_ref):
    dinv = _dinv(deg_ref)
    h = jnp.dot(x_ref[...], wp_ref[...].T, preferred_element_type=jnp.float32)
    h = h + bp_ref[...]
    g_ref[...] = dinv * jnp.dot(h, w0_ref[...].T, preferred_element_type=jnp.float32)


def _tc_mid(acc_ref, g_ref, deg_ref, b_ref, w_ref, o_ref):
    dinv = _dinv(deg_ref)
    tot = acc_ref[0] + acc_ref[1] + g_ref[...]
    h = jnp.maximum(dinv * tot + b_ref[...], 0.0)
    o_ref[...] = dinv * jnp.dot(h, w_ref[...].T, preferred_element_type=jnp.float32)


def _tc_final(acc_ref, g_ref, deg_ref, b_ref, wo_ref, bo_ref, o_ref):
    dinv = _dinv(deg_ref)
    tot = acc_ref[0] + acc_ref[1] + g_ref[...]
    h = jnp.maximum(dinv * tot + b_ref[...], 0.0)
    o_ref[...] = jnp.dot(h, wo_ref[...].T, preferred_element_type=jnp.float32)
    o_ref[...] += bo_ref[...]


def _row_spec(width):
    return pl.BlockSpec((BLK, width), lambda i: (i, 0))


def _bcast_spec(shape):
    nd = len(shape)
    return pl.BlockSpec(shape, lambda i, _n=nd: (0,) * _n)


_DEG_SPEC = pl.BlockSpec((NC, BLK, C), lambda i: (0, i, 0))
_ACC_SPEC = pl.BlockSpec((NC, BLK, C), lambda i: (0, i, 0))


def kernel(x, edge_index, Wp, bp, W0, b0, W1, b1, W2, b2, Wo, bo):
    row = edge_index[0].astype(jnp.int32)
    col = edge_index[1].astype(jnp.int32)
    npad = E_PAD - E
    rowp = jnp.concatenate([row, jnp.zeros((npad,), jnp.int32)]).reshape(TOT_CH, K)
    colp = jnp.concatenate([col, jnp.full((npad,), N, jnp.int32)]).reshape(TOT_CH, K)
    onesC = jnp.ones((K, C), jnp.float32)
    zerosC = jnp.zeros((RPT, C), jnp.float32)
    bp2 = bp.reshape(1, C)
    b0_2 = b0.reshape(1, C)
    b1_2 = b1.reshape(1, C)
    b2_2 = b2.reshape(1, C)
    bo2 = bo.reshape(1, OUTC)

    degp = _sc_degree(colp, onesC, zerosC)

    g0 = pl.pallas_call(
        _tc_first,
        grid=(NBLK,),
        in_specs=[
            _row_spec(C),
            _bcast_spec((C, C)),
            _bcast_spec((1, C)),
            _bcast_spec((C, C)),
            _DEG_SPEC,
        ],
        out_specs=_row_spec(C),
        out_shape=jax.ShapeDtypeStruct((N, C), jnp.float32),
    )(x, Wp, bp2, W0, degp)

    g = g0
    for Wn, bn in ((W1, b0_2), (W2, b1_2)):
        accp = _sc_scatter(g, rowp, colp, zerosC)
        g = pl.pallas_call(
            _tc_mid,
            grid=(NBLK,),
            in_specs=[
                _ACC_SPEC,
                _row_spec(C),
                _DEG_SPEC,
                _bcast_spec((1, C)),
                _bcast_spec((C, C)),
            ],
            out_specs=_row_spec(C),
            out_shape=jax.ShapeDtypeStruct((N, C), jnp.float32),
        )(accp, g, degp, bn, Wn)

    accp = _sc_scatter(g, rowp, colp, zerosC)
    out = pl.pallas_call(
        _tc_final,
        grid=(NBLK,),
        in_specs=[
            _ACC_SPEC,
            _row_spec(C),
            _DEG_SPEC,
            _bcast_spec((1, C)),
            _bcast_spec((OUTC, C)),
            _bcast_spec((1, OUTC)),
        ],
        out_specs=_row_spec(OUTC),
        out_shape=jax.ShapeDtypeStruct((N, OUTC), jnp.float32),
    )(accp, g, degp, b2_2, Wo, bo2)
    return out


# speedup vs baseline: 1.7232x; 1.4636x over previous
import functools

import jax
import jax.numpy as jnp
from jax import lax
from jax.experimental import pallas as pl
from jax.experimental.pallas import tpu as pltpu
from jax.experimental.pallas import tpu_sc as plsc

N = 10000
E = 320000
C = 128
OUTC = 64

NC = 2
NS = 16
NW = NC * NS
K = 128
NCHUNK = 79
EPW = NCHUNK * K
E_PAD = NW * EPW
N_PAD = 10112
RPT = N_PAD // NS

_mesh = plsc.VectorSubcoreMesh(
    core_axis_name="c", subcore_axis_name="s", num_cores=NC, num_subcores=NS
)


@functools.partial(
    pl.kernel,
    out_type=jax.ShapeDtypeStruct((NC, N_PAD, C), jnp.float32),
    mesh=_mesh,
    scratch_types=[
        pltpu.VMEM((NCHUNK, K), jnp.int32),
        pltpu.VMEM((K, C), jnp.float32),
        pltpu.VMEM_SHARED((N_PAD, C), jnp.float32),
    ],
)
def _sc_degree(col_hbm, ones_hbm, zeros_hbm, out_hbm, colv, onesv, acc):
    c = lax.axis_index("c")
    s = lax.axis_index("s")
    w = c * NS + s
    pltpu.sync_copy(zeros_hbm, acc.at[pl.ds(s * RPT, RPT)])
    pltpu.sync_copy(col_hbm.at[w], colv)
    pltpu.sync_copy(ones_hbm, onesv)
    plsc.subcore_barrier()

    def body(j, carry):
        pltpu.sync_copy(onesv, acc.at[colv.at[j]], add=True)
        return carry

    lax.fori_loop(0, NCHUNK, body, 0)
    plsc.subcore_barrier()
    pltpu.sync_copy(acc.at[pl.ds(s * RPT, RPT)], out_hbm.at[c, pl.ds(s * RPT, RPT)])


@functools.partial(
    pl.kernel,
    out_type=jax.ShapeDtypeStruct((NC, N_PAD, C), jnp.float32),
    mesh=_mesh,
    scratch_types=[
        pltpu.VMEM((NCHUNK, K), jnp.int32),
        pltpu.VMEM((NCHUNK, K), jnp.int32),
        pltpu.VMEM((K, C), jnp.float32),
        pltpu.VMEM_SHARED((N_PAD, C), jnp.float32),
        pltpu.SemaphoreType.DMA,
    ],
)
def _sc_scatter(g_hbm, row_hbm, col_hbm, zeros_hbm, out_hbm,
                rowv, colv, buf, acc, sem):
    c = lax.axis_index("c")
    s = lax.axis_index("s")
    w = c * NS + s
    pltpu.sync_copy(zeros_hbm, acc.at[pl.ds(s * RPT, RPT)])
    pltpu.sync_copy(row_hbm.at[w], rowv)
    pltpu.sync_copy(col_hbm.at[w], colv)
    plsc.subcore_barrier()

    def body(j, carry):
        pltpu.async_copy(g_hbm.at[rowv.at[j]], buf, sem).wait()
        pltpu.sync_copy(buf, acc.at[colv.at[j]], add=True)
        return carry

    lax.fori_loop(0, NCHUNK, body, 0)
    plsc.subcore_barrier()
    pltpu.sync_copy(acc.at[pl.ds(s * RPT, RPT)], out_hbm.at[c, pl.ds(s * RPT, RPT)])


BLK = 1000
NBLK = N // BLK


def _dinv(deg_ref):
    deg = deg_ref[0, :, :1] + deg_ref[1, :, :1] + 1.0
    return lax.rsqrt(deg)


def _tc_first(x_ref, wp_ref, bp_ref, w0_ref, deg_ref, g_ref):
    dinv = _dinv(deg_ref)
    h = jnp.dot(x_ref[...], wp_ref[...].T, preferred_element_type=jnp.float32)
    h = h + bp_ref[...]
    g_ref[...] = dinv * jnp.dot(h, w0_ref[...].T, preferred_element_type=jnp.float32)


def _tc_mid(acc_ref, g_ref, deg_ref, b_ref, w_ref, o_ref):
    dinv = _dinv(deg_ref)
    tot = acc_ref[0] + acc_ref[1] + g_ref[...]
    h = jnp.maximum(dinv * tot + b_ref[...], 0.0)
    o_ref[...] = dinv * jnp.dot(h, w_ref[...].T, preferred_element_type=jnp.float32)


def _tc_final(acc_ref, g_ref, deg_ref, b_ref, wo_ref, bo_ref, o_ref):
    dinv = _dinv(deg_ref)
    tot = acc_ref[0] + acc_ref[1] + g_ref[...]
    h = jnp.maximum(dinv * tot + b_ref[...], 0.0)
    o_ref[...] = jnp.dot(h, wo_ref[...].T, preferred_element_type=jnp.float32)
    o_ref[...] += bo_ref[...]


def _row_spec(width):
    return pl.BlockSpec((BLK, width), lambda i: (i, 0))


def _bcast_spec(shape):
    nd = len(shape)
    return pl.BlockSpec(shape, lambda i, _n=nd: (0,) * _n)


_DEG_SPEC = pl.BlockSpec((NC, BLK, C), lambda i: (0, i, 0))
_ACC_SPEC = pl.BlockSpec((NC, BLK, C), lambda i: (0, i, 0))


def kernel(x, edge_index, Wp, bp, W0, b0, W1, b1, W2, b2, Wo, bo):
    row = edge_index[0].astype(jnp.int32)
    col = edge_index[1].astype(jnp.int32)
    npad = E_PAD - E
    rowp = jnp.concatenate([row, jnp.zeros((npad,), jnp.int32)]).reshape(NW, NCHUNK, K)
    colp = jnp.concatenate([col, jnp.full((npad,), N, jnp.int32)]).reshape(NW, NCHUNK, K)
    onesC = jnp.ones((K, C), jnp.float32)
    zerosC = jnp.zeros((RPT, C), jnp.float32)
    bp2 = bp.reshape(1, C)
    b0_2 = b0.reshape(1, C)
    b1_2 = b1.reshape(1, C)
    b2_2 = b2.reshape(1, C)
    bo2 = bo.reshape(1, OUTC)

    degp = _sc_degree(colp, onesC, zerosC)

    g0 = pl.pallas_call(
        _tc_first,
        grid=(NBLK,),
        in_specs=[
            _row_spec(C),
            _bcast_spec((C, C)),
            _bcast_spec((1, C)),
            _bcast_spec((C, C)),
            _DEG_SPEC,
        ],
        out_specs=_row_spec(C),
        out_shape=jax.ShapeDtypeStruct((N, C), jnp.float32),
    )(x, Wp, bp2, W0, degp)

    g = g0
    for Wn, bn in ((W1, b0_2), (W2, b1_2)):
        accp = _sc_scatter(g, rowp, colp, zerosC)
        g = pl.pallas_call(
            _tc_mid,
            grid=(NBLK,),
            in_specs=[
                _ACC_SPEC,
                _row_spec(C),
                _DEG_SPEC,
                _bcast_spec((1, C)),
                _bcast_spec((C, C)),
            ],
            out_specs=_row_spec(C),
            out_shape=jax.ShapeDtypeStruct((N, C), jnp.float32),
        )(accp, g, degp, bn, Wn)

    accp = _sc_scatter(g, rowp, colp, zerosC)
    out = pl.pallas_call(
        _tc_final,
        grid=(NBLK,),
        in_specs=[
            _ACC_SPEC,
            _row_spec(C),
            _DEG_SPEC,
            _bcast_spec((1, C)),
            _bcast_spec((OUTC, C)),
            _bcast_spec((1, OUTC)),
        ],
        out_specs=_row_spec(OUTC),
        out_shape=jax.ShapeDtypeStruct((N, OUTC), jnp.float32),
    )(accp, g, degp, b2_2, Wo, bo2)
    return out


# speedup vs baseline: 1.8924x; 1.0982x over previous
import functools

import jax
import jax.numpy as jnp
from jax import lax
from jax.experimental import pallas as pl
from jax.experimental.pallas import tpu as pltpu
from jax.experimental.pallas import tpu_sc as plsc

N = 10000
E = 320000
C = 128
OUTC = 64

NC = 2
NS = 16
NW = NC * NS
K = 128
NCHUNK = 79
EPW = NCHUNK * K
E_PAD = NW * EPW
CH_A = 102
CH_B = 56
N_PAD = 10112
RPT = N_PAD // NS

_mesh = plsc.VectorSubcoreMesh(
    core_axis_name="c", subcore_axis_name="s", num_cores=NC, num_subcores=NS
)


@functools.partial(
    pl.kernel,
    out_type=jax.ShapeDtypeStruct((NC, N_PAD, C), jnp.float32),
    mesh=_mesh,
    scratch_types=[
        pltpu.VMEM((NCHUNK, K), jnp.int32),
        pltpu.VMEM((K, C), jnp.float32),
        pltpu.VMEM_SHARED((N_PAD, C), jnp.float32),
    ],
)
def _sc_degree(col_hbm, ones_hbm, zeros_hbm, out_hbm, colv, onesv, acc):
    c = lax.axis_index("c")
    s = lax.axis_index("s")
    w = c * NS + s
    pltpu.sync_copy(zeros_hbm, acc.at[pl.ds(s * RPT, RPT)])
    pltpu.sync_copy(col_hbm.at[w], colv)
    pltpu.sync_copy(ones_hbm, onesv)
    plsc.subcore_barrier()

    def body(j, carry):
        pltpu.sync_copy(onesv, acc.at[colv.at[j]], add=True)
        return carry

    lax.fori_loop(0, NCHUNK, body, 0)
    plsc.subcore_barrier()
    pltpu.sync_copy(acc.at[pl.ds(s * RPT, RPT)], out_hbm.at[c, pl.ds(s * RPT, RPT)])


@functools.partial(
    pl.kernel,
    out_type=jax.ShapeDtypeStruct((NC, N_PAD, C), jnp.float32),
    mesh=_mesh,
    scratch_types=[
        pltpu.VMEM((CH_A, K), jnp.int32),
        pltpu.VMEM((CH_A, K), jnp.int32),
        pltpu.VMEM((K, C), jnp.float32),
        pltpu.VMEM_SHARED((N_PAD, C), jnp.float32),
        pltpu.SemaphoreType.DMA,
    ],
)
def _sc_scatter(g_hbm, rowa_hbm, cola_hbm, rowb_hbm, colb_hbm, zeros_hbm,
                out_hbm, rowv, colv, buf, acc, sem):
    c = lax.axis_index("c")
    s = lax.axis_index("s")
    pltpu.sync_copy(zeros_hbm, acc.at[pl.ds(s * RPT, RPT)])
    plsc.subcore_barrier()

    def body(j, carry):
        pltpu.async_copy(g_hbm.at[rowv.at[j]], buf, sem).wait()
        pltpu.sync_copy(buf, acc.at[colv.at[j]], add=True)
        return carry

    @pl.when(c == 0)
    def _():
        pltpu.sync_copy(rowa_hbm.at[s], rowv)
        pltpu.sync_copy(cola_hbm.at[s], colv)
        lax.fori_loop(0, CH_A, body, 0)

    @pl.when(c == 1)
    def _():
        pltpu.sync_copy(rowb_hbm.at[s], rowv.at[pl.ds(0, CH_B)])
        pltpu.sync_copy(colb_hbm.at[s], colv.at[pl.ds(0, CH_B)])
        lax.fori_loop(0, CH_B, body, 0)

    plsc.subcore_barrier()
    pltpu.sync_copy(acc.at[pl.ds(s * RPT, RPT)], out_hbm.at[c, pl.ds(s * RPT, RPT)])


BLK = 1000
NBLK = N // BLK


def _dinv(deg_ref):
    deg = deg_ref[0, :, :1] + deg_ref[1, :, :1] + 1.0
    return lax.rsqrt(deg)


def _tc_first(x_ref, wp_ref, bp_ref, w0_ref, deg_ref, g_ref):
    dinv = _dinv(deg_ref)
    h = jnp.dot(x_ref[...], wp_ref[...].T, preferred_element_type=jnp.float32)
    h = h + bp_ref[...]
    g_ref[...] = dinv * jnp.dot(h, w0_ref[...].T, preferred_element_type=jnp.float32)


def _tc_mid(acc_ref, g_ref, deg_ref, b_ref, w_ref, o_ref):
    dinv = _dinv(deg_ref)
    tot = acc_ref[0] + acc_ref[1] + g_ref[...]
    h = jnp.maximum(dinv * tot + b_ref[...], 0.0)
    o_ref[...] = dinv * jnp.dot(h, w_ref[...].T, preferred_element_type=jnp.float32)


def _tc_final(acc_ref, g_ref, deg_ref, b_ref, wo_ref, bo_ref, o_ref):
    dinv = _dinv(deg_ref)
    tot = acc_ref[0] + acc_ref[1] + g_ref[...]
    h = jnp.maximum(dinv * tot + b_ref[...], 0.0)
    o_ref[...] = jnp.dot(h, wo_ref[...].T, preferred_element_type=jnp.float32)
    o_ref[...] += bo_ref[...]


def _row_spec(width):
    return pl.BlockSpec((BLK, width), lambda i: (i, 0))


def _bcast_spec(shape):
    nd = len(shape)
    return pl.BlockSpec(shape, lambda i, _n=nd: (0,) * _n)


_DEG_SPEC = pl.BlockSpec((NC, BLK, C), lambda i: (0, i, 0))
_ACC_SPEC = pl.BlockSpec((NC, BLK, C), lambda i: (0, i, 0))


def kernel(x, edge_index, Wp, bp, W0, b0, W1, b1, W2, b2, Wo, bo):
    row = edge_index[0].astype(jnp.int32)
    col = edge_index[1].astype(jnp.int32)
    npad = E_PAD - E
    rowf = jnp.concatenate([row, jnp.zeros((npad,), jnp.int32)])
    colf = jnp.concatenate([col, jnp.full((npad,), N, jnp.int32)])
    colp = colf.reshape(NW, NCHUNK, K)
    na = NS * CH_A * K
    rowa = rowf[:na].reshape(NS, CH_A, K)
    cola = colf[:na].reshape(NS, CH_A, K)
    rowb = rowf[na:].reshape(NS, CH_B, K)
    colb = colf[na:].reshape(NS, CH_B, K)
    onesC = jnp.ones((K, C), jnp.float32)
    zerosC = jnp.zeros((RPT, C), jnp.float32)
    bp2 = bp.reshape(1, C)
    b0_2 = b0.reshape(1, C)
    b1_2 = b1.reshape(1, C)
    b2_2 = b2.reshape(1, C)
    bo2 = bo.reshape(1, OUTC)

    degp = _sc_degree(colp, onesC, zerosC)

    g0 = pl.pallas_call(
        _tc_first,
        grid=(NBLK,),
        in_specs=[
            _row_spec(C),
            _bcast_spec((C, C)),
            _bcast_spec((1, C)),
            _bcast_spec((C, C)),
            _DEG_SPEC,
        ],
        out_specs=_row_spec(C),
        out_shape=jax.ShapeDtypeStruct((N, C), jnp.float32),
    )(x, Wp, bp2, W0, degp)

    g = g0
    for Wn, bn in ((W1, b0_2), (W2, b1_2)):
        accp = _sc_scatter(g, rowa, cola, rowb, colb, zerosC)
        g = pl.pallas_call(
            _tc_mid,
            grid=(NBLK,),
            in_specs=[
                _ACC_SPEC,
                _row_spec(C),
                _DEG_SPEC,
                _bcast_spec((1, C)),
                _bcast_spec((C, C)),
            ],
            out_specs=_row_spec(C),
            out_shape=jax.ShapeDtypeStruct((N, C), jnp.float32),
        )(accp, g, degp, bn, Wn)

    accp = _sc_scatter(g, rowa, cola, rowb, colb, zerosC)
    out = pl.pallas_call(
        _tc_final,
        grid=(NBLK,),
        in_specs=[
            _ACC_SPEC,
            _row_spec(C),
            _DEG_SPEC,
            _bcast_spec((1, C)),
            _bcast_spec((OUTC, C)),
            _bcast_spec((1, OUTC)),
        ],
        out_specs=_row_spec(OUTC),
        out_shape=jax.ShapeDtypeStruct((N, OUTC), jnp.float32),
    )(accp, g, degp, b2_2, Wo, bo2)
    return out
